# Initial kernel scaffold; baseline (speedup 1.0000x reference)
import functools
import numpy as np
import jax
import jax.numpy as jnp
from jax import lax
from jax.experimental import pallas as pl
from jax.experimental.pallas import tpu as pltpu
from jax.experimental.pallas import tpu_sc as plsc

N = 10000
E = 320000
D = 128
C = 32
DIR = 4
NW = 64
DMID = 256
AVG_NEIGH = 32.0

NT = 16
EB = 80
EPT = E // NT
NCH = EPT // EB
ROWS_PT = N // NT
ZR = 125

_PCOL_UP = np.array([(t % 32) * 4 + t // 32 for t in range(D)], dtype=np.int32)
_PROW_DN = np.array(
    [((t // 128) * 32 + t % 32) * 4 + (t % 128) // 32 for t in range(DMID)],
    dtype=np.int32,
)
_QCOL = np.concatenate(
    [np.arange(128), 128 + 2 * np.arange(64), 129 + 2 * np.arange(64)]
).astype(np.int32)


def _up_body(nf_ref, w_ref, o_ref):
    o_ref[...] = jnp.dot(nf_ref[...], w_ref[...],
                         preferred_element_type=jnp.float32)


def _linear_up(nf, w_up_p):
    bn = 1000
    return pl.pallas_call(
        _up_body,
        grid=(N // bn,),
        in_specs=[
            pl.BlockSpec((bn, D), lambda i: (i, 0)),
            pl.BlockSpec((D, D), lambda i: (0, 0)),
        ],
        out_specs=pl.BlockSpec((bn, D), lambda i: (i, 0)),
        out_shape=jax.ShapeDtypeStruct((N, D), jnp.float32),
    )(nf, w_up_p)


def _edge_prep_body(ele_ref, ef_ref, wr1_ref, wr2_ref, cg2_ref, w_ref, m_ref):
    hid = jnp.dot(ele_ref[...], wr1_ref[...],
                  preferred_element_type=jnp.float32)
    hid = hid * jax.nn.sigmoid(hid)
    w_ref[...] = jnp.dot(hid, wr2_ref[...],
                         preferred_element_type=jnp.float32)
    m_ref[...] = jnp.dot(ef_ref[...], cg2_ref[...],
                         preferred_element_type=jnp.float32)


def _edge_prep(ele, ef, w_r1, w_r2, cg2):
    be = 8000
    return pl.pallas_call(
        _edge_prep_body,
        grid=(E // be,),
        in_specs=[
            pl.BlockSpec((be, 8), lambda i: (i, 0)),
            pl.BlockSpec((be, 4), lambda i: (i, 0)),
            pl.BlockSpec((8, 8), lambda i: (0, 0)),
            pl.BlockSpec((8, NW), lambda i: (0, 0)),
            pl.BlockSpec((4, 16), lambda i: (0, 0)),
        ],
        out_specs=[
            pl.BlockSpec((be, NW), lambda i: (i, 0)),
            pl.BlockSpec((be, 16), lambda i: (i, 0)),
        ],
        out_shape=[
            jax.ShapeDtypeStruct((E, NW), jnp.float32),
            jax.ShapeDtypeStruct((E, 16), jnp.float32),
        ],
    )(ele, ef, w_r1, w_r2, cg2)


@functools.partial(
    pl.kernel,
    mesh=plsc.VectorSubcoreMesh(core_axis_name="c", subcore_axis_name="s"),
    out_type=jax.ShapeDtypeStruct((2, N, 128), jnp.float32),
    scratch_types=[
        pltpu.VMEM((EB,), jnp.int32),
        pltpu.VMEM((EB,), jnp.int32),
        pltpu.VMEM((EB, 128), jnp.float32),
        pltpu.VMEM((EB, 32), jnp.float32),
        pltpu.VMEM((EB, 16), jnp.float32),
        pltpu.VMEM((EB, 128), jnp.float32),
        pltpu.VMEM((ZR, 128), jnp.float32),
        pltpu.VMEM_SHARED((N, 128), jnp.float32),
        pltpu.SemaphoreType.DMA,
    ],
)
def _edge_kernel(h_hbm, snd_hbm, rcv_hbm, w_hbm, m_hbm, out_hbm,
                 snd_v, rcv_v, msg_v, w_v, m_v, o_v, z_v, agg_sh, sem):
    cid = lax.axis_index("c")
    sid = lax.axis_index("s")

    zero16 = jnp.zeros((16,), jnp.float32)

    def zrow(r, carry):
        for v in range(8):
            z_v[r, pl.ds(v * 16, 16)] = zero16
        return carry

    lax.fori_loop(0, ZR, zrow, 0)
    rbase = sid * ROWS_PT
    for zi in range(ROWS_PT // ZR):
        pltpu.sync_copy(z_v, agg_sh.at[pl.ds(rbase + zi * ZR, ZR)])
    plsc.subcore_barrier()

    ebase = sid * EPT

    def chunk(ch, carry):
        eb = ebase + ch * EB
        pltpu.sync_copy(snd_hbm.at[pl.ds(eb, EB)], snd_v)
        pltpu.sync_copy(rcv_hbm.at[pl.ds(eb, EB)], rcv_v)
        pltpu.sync_copy(w_hbm.at[pl.ds(eb, EB), pl.ds(cid * 32, 32)], w_v)

        @pl.when(cid == 1)
        def _():
            pltpu.sync_copy(m_hbm.at[pl.ds(eb, EB)], m_v)

        pltpu.async_copy(h_hbm.at[snd_v], msg_v, sem).wait()

        @pl.when(cid == 0)
        def _():
            def edge0(j, c0):
                w0 = w_v[j, pl.ds(0, 16)]
                w1 = w_v[j, pl.ds(16, 16)]
                for v in range(8):
                    wv = w0 if v % 2 == 0 else w1
                    o_v[j, pl.ds(v * 16, 16)] = msg_v[j, pl.ds(v * 16, 16)] * wv
                return c0

            lax.fori_loop(0, EB, edge0, 0)

        @pl.when(cid == 1)
        def _():
            def edge1(j, c1):
                w0 = w_v[j, pl.ds(0, 16)]
                w1 = w_v[j, pl.ds(16, 16)]
                mrow = [msg_v[j, pl.ds(v * 16, 16)] for v in range(8)]
                msp = [[jnp.broadcast_to(m_v[j, i * 4 + k], (16,))
                        for k in range(4)] for i in range(4)]
                for k in range(4):
                    for h in range(2):
                        acc = mrow[0 * 2 + h] * msp[0][k]
                        for i in range(1, 4):
                            acc = acc + mrow[i * 2 + h] * msp[i][k]
                        wv = w0 if h == 0 else w1
                        o_v[j, pl.ds((k * 2 + h) * 16, 16)] = acc * wv
                return c1

            lax.fori_loop(0, EB, edge1, 0)

        pltpu.sync_copy(o_v, agg_sh.at[rcv_v], add=True)
        return carry

    lax.fori_loop(0, NCH, chunk, 0)
    plsc.subcore_barrier()
    pltpu.sync_copy(agg_sh.at[pl.ds(rbase, ROWS_PT)],
                    out_hbm.at[cid, pl.ds(rbase, ROWS_PT)])


def _final_body(agg_ref, wd_ref, wsc_ref, oh_ref, o_ref):
    flat = agg_ref[...] * jnp.float32(1.0 / np.sqrt(AVG_NEIGH))
    hdn = jnp.dot(flat, wd_ref[...], preferred_element_type=jnp.float32)
    sc = oh_ref[:, 0:1] * jnp.dot(hdn, wsc_ref[0],
                                  preferred_element_type=jnp.float32)
    for s in range(1, 4):
        sc = sc + oh_ref[:, s:s + 1] * jnp.dot(
            hdn, wsc_ref[s], preferred_element_type=jnp.float32)
    hdn = hdn + sc
    scal = hdn[:, :NW]
    gates = hdn[:, NW:2 * NW]
    gsil = gates * jax.nn.sigmoid(gates)
    o_ref[:, :NW] = scal * jax.nn.sigmoid(scal)
    o_ref[:, NW:2 * NW] = hdn[:, 2 * NW:3 * NW] * gsil
    o_ref[:, 2 * NW:3 * NW] = hdn[:, 3 * NW:4 * NW] * gsil


def _finalize(agg_t, wd_pq, wsc_q, onehot):
    bn = 1000
    return pl.pallas_call(
        _final_body,
        grid=(N // bn,),
        in_specs=[
            pl.BlockSpec((bn, DMID), lambda i: (i, 0)),
            pl.BlockSpec((DMID, DMID), lambda i: (0, 0)),
            pl.BlockSpec((4, DMID, DMID), lambda i: (0, 0, 0)),
            pl.BlockSpec((bn, 4), lambda i: (i, 0)),
        ],
        out_specs=pl.BlockSpec((bn, 3 * NW), lambda i: (i, 0)),
        out_shape=jax.ShapeDtypeStruct((N, 3 * NW), jnp.float32),
    )(agg_t, wd_pq, wsc_q, onehot)


def kernel(node_features, edge_features, edge_length_embeddings, senders,
           receivers, node_species, W_up, cg, W_r1, W_r2, W_down, W_sc):
    pcol = jnp.asarray(_PCOL_UP)
    prow = jnp.asarray(_PROW_DN)
    qcol = jnp.asarray(_QCOL)

    w_up_p = W_up[:, pcol]
    cg2 = jnp.transpose(cg, (1, 0, 2)).reshape(4, 16)
    wd_pq = W_down[prow][:, qcol]
    wsc_q = W_sc[:, qcol][:, :, qcol]
    onehot = (node_species[:, None] == jnp.arange(4)[None, :]).astype(
        jnp.float32)

    h_t = _linear_up(node_features, w_up_p)
    w_e, m_e = _edge_prep(edge_length_embeddings, edge_features, W_r1, W_r2,
                          cg2)
    agg2 = _edge_kernel(h_t, senders.astype(jnp.int32),
                        receivers.astype(jnp.int32), w_e, m_e)
    agg_t = jnp.concatenate([agg2[0], agg2[1]], axis=1)

    o = _finalize(agg_t, wd_pq, wsc_q, onehot)
    out = jnp.concatenate(
        [o[:, :NW],
         jnp.stack([o[:, NW:2 * NW], o[:, 2 * NW:]], axis=-1).reshape(
             N, 2 * NW)],
        axis=1)
    return out


# speedup vs baseline: 9.5171x; 9.5171x over previous
import functools
import numpy as np
import jax
import jax.numpy as jnp
from jax import lax
from jax.experimental import pallas as pl
from jax.experimental.pallas import tpu as pltpu
from jax.experimental.pallas import tpu_sc as plsc

N = 10000
E = 320000
D = 128
C = 32
DIR = 4
NW = 64
DMID = 256
AVG_NEIGH = 32.0

NT = 16
EB = 80
EPT = E // NT
NCH = EPT // EB
NPAD = 10240
ROWS_PT = NPAD // NT
ZR = 128

_PCOL_UP = np.array([(t % 32) * 4 + t // 32 for t in range(D)], dtype=np.int32)
_PROW_DN = np.array(
    [((t // 128) * 32 + t % 32) * 4 + (t % 128) // 32 for t in range(DMID)],
    dtype=np.int32,
)
_QCOL = np.concatenate(
    [np.arange(128), 128 + 2 * np.arange(64), 129 + 2 * np.arange(64)]
).astype(np.int32)


def _up_body(nf_ref, w_ref, o_ref):
    o_ref[...] = jnp.dot(nf_ref[...], w_ref[...],
                         preferred_element_type=jnp.float32)


def _linear_up(nf, w_up_p):
    bn = 1000
    return pl.pallas_call(
        _up_body,
        grid=(N // bn,),
        in_specs=[
            pl.BlockSpec((bn, D), lambda i: (i, 0)),
            pl.BlockSpec((D, D), lambda i: (0, 0)),
        ],
        out_specs=pl.BlockSpec((bn, D), lambda i: (i, 0)),
        out_shape=jax.ShapeDtypeStruct((N, D), jnp.float32),
    )(nf, w_up_p)


def _edge_prep_body(ele_ref, ef_ref, wr1_ref, wr2_ref, cg2_ref, w_ref, m_ref):
    hid = jnp.dot(ele_ref[...], wr1_ref[...],
                  preferred_element_type=jnp.float32)
    hid = hid * jax.nn.sigmoid(hid)
    w = jnp.dot(hid, wr2_ref[...], preferred_element_type=jnp.float32)
    w_ref[0] = w[:, :32]
    w_ref[1] = w[:, 32:]
    m_ref[...] = jnp.dot(ef_ref[...], cg2_ref[...],
                         preferred_element_type=jnp.float32)


def _edge_prep(ele, ef, w_r1, w_r2, cg2):
    be = 8000
    return pl.pallas_call(
        _edge_prep_body,
        grid=(E // be,),
        in_specs=[
            pl.BlockSpec((be, 8), lambda i: (i, 0)),
            pl.BlockSpec((be, 4), lambda i: (i, 0)),
            pl.BlockSpec((8, 8), lambda i: (0, 0)),
            pl.BlockSpec((8, NW), lambda i: (0, 0)),
            pl.BlockSpec((4, 16), lambda i: (0, 0)),
        ],
        out_specs=[
            pl.BlockSpec((2, be, 32), lambda i: (0, i, 0)),
            pl.BlockSpec((be, 16), lambda i: (i, 0)),
        ],
        out_shape=[
            jax.ShapeDtypeStruct((2, E, 32), jnp.float32),
            jax.ShapeDtypeStruct((E, 16), jnp.float32),
        ],
    )(ele, ef, w_r1, w_r2, cg2)


@functools.lru_cache(maxsize=None)
def _make_edge_kernel():
    return functools.partial(
        pl.kernel,
        mesh=plsc.VectorSubcoreMesh(core_axis_name="c", subcore_axis_name="s"),
        out_type=jax.ShapeDtypeStruct((2, NPAD, 128), jnp.float32),
        scratch_types=[
            pltpu.VMEM((EB,), jnp.int32),
            pltpu.VMEM((EB,), jnp.int32),
            pltpu.VMEM((EB, 128), jnp.float32),
            pltpu.VMEM((EB * 32,), jnp.float32),
            pltpu.VMEM((EB * 16,), jnp.float32),
            pltpu.VMEM((EB, 128), jnp.float32),
            pltpu.VMEM((ZR, 128), jnp.float32),
            pltpu.VMEM_SHARED((NPAD, 128), jnp.float32),
            pltpu.SemaphoreType.DMA,
        ],
    )(_edge_body)


def _edge_body(h_hbm, snd_hbm, rcv_hbm, w_hbm, m_hbm, out_hbm,
               snd_v, rcv_v, msg_v, w_v, m_v, o_v, z_v, agg_sh, sem):
    cid = lax.axis_index("c")
    sid = lax.axis_index("s")

    zero16 = jnp.zeros((16,), jnp.float32)

    def zrow(r, carry):
        for v in range(8):
            z_v[r, pl.ds(v * 16, 16)] = zero16
        return carry

    lax.fori_loop(0, ZR, zrow, 0)
    rbase = sid * ROWS_PT
    for zi in range(ROWS_PT // ZR):
        pltpu.sync_copy(z_v, agg_sh.at[pl.ds(rbase + zi * ZR, ZR)])
    plsc.subcore_barrier()

    ebase = sid * EPT

    def chunk(ch, carry):
        eb = ebase + ch * EB
        pltpu.sync_copy(snd_hbm.at[pl.ds(eb, EB)], snd_v)
        pltpu.sync_copy(rcv_hbm.at[pl.ds(eb, EB)], rcv_v)
        pltpu.sync_copy(w_hbm.at[cid, pl.ds(eb * 32, EB * 32)], w_v)

        @pl.when(cid == 1)
        def _():
            pltpu.sync_copy(m_hbm.at[pl.ds(eb * 16, EB * 16)], m_v)

        pltpu.async_copy(h_hbm.at[snd_v], msg_v, sem).wait()

        @pl.when(cid == 0)
        def _():
            def edge0(j, c0):
                w0 = w_v[pl.ds(j * 32, 16)]
                w1 = w_v[pl.ds(j * 32 + 16, 16)]
                for v in range(8):
                    wv = w0 if v % 2 == 0 else w1
                    o_v[j, pl.ds(v * 16, 16)] = msg_v[j, pl.ds(v * 16, 16)] * wv
                return c0

            lax.fori_loop(0, EB, edge0, 0)

        @pl.when(cid == 1)
        def _():
            def edge1(j, c1):
                w0 = w_v[pl.ds(j * 32, 16)]
                w1 = w_v[pl.ds(j * 32 + 16, 16)]
                mrow = [msg_v[j, pl.ds(v * 16, 16)] for v in range(8)]
                m16 = m_v[pl.ds(j * 16, 16)]
                msp = [[jnp.broadcast_to(m16[i * 4 + k], (16,))
                        for k in range(4)] for i in range(4)]
                for k in range(4):
                    for h in range(2):
                        acc = mrow[0 * 2 + h] * msp[0][k]
                        for i in range(1, 4):
                            acc = acc + mrow[i * 2 + h] * msp[i][k]
                        wv = w0 if h == 0 else w1
                        o_v[j, pl.ds((k * 2 + h) * 16, 16)] = acc * wv
                return c1

            lax.fori_loop(0, EB, edge1, 0)

        pltpu.sync_copy(o_v, agg_sh.at[rcv_v], add=True)
        return carry

    lax.fori_loop(0, NCH, chunk, 0)
    plsc.subcore_barrier()
    pltpu.sync_copy(agg_sh.at[pl.ds(rbase, ROWS_PT)],
                    out_hbm.at[cid, pl.ds(rbase, ROWS_PT)])


def _final_body(agg_ref, wd_ref, wsc_ref, oh_ref, o_ref):
    flat = agg_ref[...] * jnp.float32(1.0 / np.sqrt(AVG_NEIGH))
    hdn = jnp.dot(flat, wd_ref[...], preferred_element_type=jnp.float32)
    sc = oh_ref[:, 0:1] * jnp.dot(hdn, wsc_ref[0],
                                  preferred_element_type=jnp.float32)
    for s in range(1, 4):
        sc = sc + oh_ref[:, s:s + 1] * jnp.dot(
            hdn, wsc_ref[s], preferred_element_type=jnp.float32)
    hdn = hdn + sc
    scal = hdn[:, :NW]
    gates = hdn[:, NW:2 * NW]
    gsil = gates * jax.nn.sigmoid(gates)
    o_ref[:, :NW] = scal * jax.nn.sigmoid(scal)
    o_ref[:, NW:2 * NW] = hdn[:, 2 * NW:3 * NW] * gsil
    o_ref[:, 2 * NW:3 * NW] = hdn[:, 3 * NW:4 * NW] * gsil


def _finalize(agg_t, wd_pq, wsc_q, onehot):
    bn = 1000
    return pl.pallas_call(
        _final_body,
        grid=(N // bn,),
        in_specs=[
            pl.BlockSpec((bn, DMID), lambda i: (i, 0)),
            pl.BlockSpec((DMID, DMID), lambda i: (0, 0)),
            pl.BlockSpec((4, DMID, DMID), lambda i: (0, 0, 0)),
            pl.BlockSpec((bn, 4), lambda i: (i, 0)),
        ],
        out_specs=pl.BlockSpec((bn, 3 * NW), lambda i: (i, 0)),
        out_shape=jax.ShapeDtypeStruct((N, 3 * NW), jnp.float32),
    )(agg_t, wd_pq, wsc_q, onehot)


def _perm_cols_t(w):
    return w.reshape(-1, 32, 4).transpose(0, 2, 1).reshape(w.shape[0], 128)


def _perm_rows_dn(w):
    return w.reshape(2, 32, 4, w.shape[1]).transpose(0, 2, 1, 3).reshape(
        w.shape[0], w.shape[1])


def _perm_cols_q(w):
    ns = w[..., 128:].reshape(*w.shape[:-1], 64, 2)
    ns = jnp.swapaxes(ns, -1, -2).reshape(*w.shape[:-1], 128)
    return jnp.concatenate([w[..., :128], ns], axis=-1)


def _perm_rows_q(w):
    ns = w[..., 128:, :].reshape(*w.shape[:-2], 64, 2, w.shape[-1])
    ns = jnp.swapaxes(ns, -2, -3).reshape(*w.shape[:-2], 128, w.shape[-1])
    return jnp.concatenate([w[..., :128, :], ns], axis=-2)


def kernel(node_features, edge_features, edge_length_embeddings, senders,
           receivers, node_species, W_up, cg, W_r1, W_r2, W_down, W_sc):
    w_up_p = _perm_cols_t(W_up)
    cg2 = jnp.transpose(cg, (1, 0, 2)).reshape(4, 16)
    wd_pq = _perm_cols_q(_perm_rows_dn(W_down))
    wsc_q = _perm_cols_q(_perm_rows_q(W_sc))
    onehot = (node_species[:, None] == jnp.arange(4)[None, :]).astype(
        jnp.float32)

    h_t = _linear_up(node_features, w_up_p)
    w_e, m_e = _edge_prep(edge_length_embeddings, edge_features, W_r1, W_r2,
                          cg2)
    agg2 = _make_edge_kernel()(h_t, senders.astype(jnp.int32),
                               receivers.astype(jnp.int32),
                               w_e.reshape(2, E * 32), m_e.reshape(E * 16))
    agg_t = jnp.concatenate([agg2[0, :N], agg2[1, :N]], axis=1)

    o = _finalize(agg_t, wd_pq, wsc_q, onehot)
    out = jnp.concatenate(
        [o[:, :NW],
         jnp.stack([o[:, NW:2 * NW], o[:, 2 * NW:]], axis=-1).reshape(
             N, 2 * NW)],
        axis=1)
    return out


# speedup vs baseline: 23.1304x; 2.4304x over previous
import functools
import numpy as np
import jax
import jax.numpy as jnp
from jax import lax
from jax.experimental import pallas as pl
from jax.experimental.pallas import tpu as pltpu
from jax.experimental.pallas import tpu_sc as plsc

N = 10000
E = 320000
D = 128
C = 32
DIR = 4
NW = 64
DMID = 256
AVG_NEIGH = 32.0

NT = 16
EB = 80
EPT = E // NT
NCH = EPT // EB
NPAD = 10112
ROWS_PT = NPAD // NT
ZR = 79

_PCOL_UP = np.array([(t % 32) * 4 + t // 32 for t in range(D)], dtype=np.int32)
_PROW_DN = np.array(
    [((t // 128) * 32 + t % 32) * 4 + (t % 128) // 32 for t in range(DMID)],
    dtype=np.int32,
)
_QCOL = np.concatenate(
    [np.arange(128), 128 + 2 * np.arange(64), 129 + 2 * np.arange(64)]
).astype(np.int32)


def _up_body(nf_ref, w_ref, o_ref):
    o_ref[...] = jnp.dot(nf_ref[...], w_ref[...],
                         preferred_element_type=jnp.float32)


def _linear_up(nf, w_up_p):
    bn = 1000
    return pl.pallas_call(
        _up_body,
        grid=(N // bn,),
        in_specs=[
            pl.BlockSpec((bn, D), lambda i: (i, 0)),
            pl.BlockSpec((D, D), lambda i: (0, 0)),
        ],
        out_specs=pl.BlockSpec((bn, D), lambda i: (i, 0)),
        out_shape=jax.ShapeDtypeStruct((N, D), jnp.float32),
    )(nf, w_up_p)


def _edge_prep_body(ele_ref, ef_ref, wr1_ref, wr2_ref, cg2_ref, p_ref):
    hid = jnp.dot(ele_ref[...], wr1_ref[...],
                  preferred_element_type=jnp.float32)
    hid = hid * jax.nn.sigmoid(hid)
    w = jnp.dot(hid, wr2_ref[...], preferred_element_type=jnp.float32)
    m = jnp.dot(ef_ref[...], cg2_ref[...], preferred_element_type=jnp.float32)
    p_ref[...] = jnp.concatenate([w, m], axis=1)


def _edge_prep(ele, ef, w_r1, w_r2, cg2):
    be = 8000
    return pl.pallas_call(
        _edge_prep_body,
        grid=(E // be,),
        in_specs=[
            pl.BlockSpec((be, 8), lambda i: (i, 0)),
            pl.BlockSpec((be, 4), lambda i: (i, 0)),
            pl.BlockSpec((8, 8), lambda i: (0, 0)),
            pl.BlockSpec((8, NW), lambda i: (0, 0)),
            pl.BlockSpec((4, 16), lambda i: (0, 0)),
        ],
        out_specs=pl.BlockSpec((be, 80), lambda i: (i, 0)),
        out_shape=jax.ShapeDtypeStruct((E, 80), jnp.float32),
    )(ele, ef, w_r1, w_r2, cg2)


@functools.lru_cache(maxsize=None)
def _make_edge_kernel():
    return functools.partial(
        pl.kernel,
        mesh=plsc.VectorSubcoreMesh(core_axis_name="c", subcore_axis_name="s"),
        out_type=jax.ShapeDtypeStruct((2, NPAD, 128), jnp.float32),
        scratch_types=[
            pltpu.VMEM((2, EB), jnp.int32),
            pltpu.VMEM((EB,), jnp.int32),
            pltpu.VMEM((2, EB, 128), jnp.float32),
            pltpu.VMEM((EB * 80,), jnp.float32),
            pltpu.VMEM((EB, 128), jnp.float32),
            pltpu.VMEM((ZR, 128), jnp.float32),
            pltpu.VMEM_SHARED((NPAD, 128), jnp.float32),
            pltpu.SemaphoreType.DMA,
        ],
    )(_edge_body)


def _edge_body(h_hbm, snd_hbm, rcv_hbm, p_hbm, out_hbm,
               snd2_v, rcv_v, msg2_v, p_v, o_v, z_v, agg_sh, sem):
    cid = lax.axis_index("c")
    sid = lax.axis_index("s")

    zero16 = jnp.zeros((16,), jnp.float32)

    def zrow(r, carry):
        for v in range(8):
            z_v[r, pl.ds(v * 16, 16)] = zero16
        return carry

    lax.fori_loop(0, ZR, zrow, 0)
    rbase = sid * ROWS_PT
    for zi in range(ROWS_PT // ZR):
        pltpu.sync_copy(z_v, agg_sh.at[pl.ds(rbase + zi * ZR, ZR)])
    plsc.subcore_barrier()

    ebase = sid * EPT

    pltpu.sync_copy(snd_hbm.at[pl.ds(ebase, EB)], snd2_v.at[0])
    pltpu.async_copy(h_hbm.at[snd2_v.at[0]], msg2_v.at[0], sem)

    def chunk(ch, carry):
        par = lax.rem(ch, 2)
        nxt = 1 - par
        eb = ebase + ch * EB
        pltpu.make_async_copy(h_hbm.at[snd2_v.at[par]], msg2_v.at[par],
                              sem).wait()

        @pl.when(ch + 1 < NCH)
        def _():
            ebn = eb + EB
            pltpu.sync_copy(snd_hbm.at[pl.ds(ebn, EB)], snd2_v.at[nxt])
            pltpu.async_copy(h_hbm.at[snd2_v.at[nxt]], msg2_v.at[nxt], sem)

        pltpu.sync_copy(rcv_hbm.at[pl.ds(eb, EB)], rcv_v)
        pltpu.sync_copy(p_hbm.at[pl.ds(eb * 80, EB * 80)], p_v)

        def edge(j, cc):
            w1a = p_v[pl.ds(j * 80, 16)]
            w1b = p_v[pl.ds(j * 80 + 16, 16)]
            w2a = p_v[pl.ds(j * 80 + 32, 16)]
            w2b = p_v[pl.ds(j * 80 + 48, 16)]
            m8 = p_v[pl.ds(j * 80 + 64 + cid * 8, 16)]
            mrow = [msg2_v[par, j, pl.ds(v * 16, 16)] for v in range(8)]
            for u in range(4):
                src = msg2_v[par, j,
                             pl.ds(cid * 64 + (u // 2) * 32 + (u % 2) * 16, 16)]
                wv = w1a if u % 2 == 0 else w1b
                o_v[j, pl.ds(u * 16, 16)] = src * wv
            for u in range(4, 8):
                kl = (u - 4) // 2
                h = u % 2
                acc = mrow[0 * 2 + h] * jnp.broadcast_to(m8[kl * 4 + 0], (16,))
                for i in range(1, 4):
                    acc = acc + mrow[i * 2 + h] * jnp.broadcast_to(
                        m8[kl * 4 + i], (16,))
                wv = w2a if h == 0 else w2b
                o_v[j, pl.ds(u * 16, 16)] = acc * wv
            return cc

        lax.fori_loop(0, EB, edge, 0)
        pltpu.sync_copy(o_v, agg_sh.at[rcv_v], add=True)
        return carry

    lax.fori_loop(0, NCH, chunk, 0)
    plsc.subcore_barrier()
    pltpu.sync_copy(agg_sh.at[pl.ds(rbase, ROWS_PT)],
                    out_hbm.at[cid, pl.ds(rbase, ROWS_PT)])


def _final_body(agg_ref, wd_ref, wsc_ref, oh_ref, o_ref):
    flat = agg_ref[...] * jnp.float32(1.0 / np.sqrt(AVG_NEIGH))
    hdn = jnp.dot(flat, wd_ref[...], preferred_element_type=jnp.float32)
    sc = oh_ref[:, 0:1] * jnp.dot(hdn, wsc_ref[0],
                                  preferred_element_type=jnp.float32)
    for s in range(1, 4):
        sc = sc + oh_ref[:, s:s + 1] * jnp.dot(
            hdn, wsc_ref[s], preferred_element_type=jnp.float32)
    hdn = hdn + sc
    scal = hdn[:, :NW]
    gates = hdn[:, NW:2 * NW]
    gsil = gates * jax.nn.sigmoid(gates)
    o_ref[:, :NW] = scal * jax.nn.sigmoid(scal)
    o_ref[:, NW:2 * NW] = hdn[:, 2 * NW:3 * NW] * gsil
    o_ref[:, 2 * NW:3 * NW] = hdn[:, 3 * NW:4 * NW] * gsil


def _finalize(agg_t, wd_pq, wsc_q, onehot):
    bn = 1000
    return pl.pallas_call(
        _final_body,
        grid=(N // bn,),
        in_specs=[
            pl.BlockSpec((bn, DMID), lambda i: (i, 0)),
            pl.BlockSpec((DMID, DMID), lambda i: (0, 0)),
            pl.BlockSpec((4, DMID, DMID), lambda i: (0, 0, 0)),
            pl.BlockSpec((bn, 4), lambda i: (i, 0)),
        ],
        out_specs=pl.BlockSpec((bn, 3 * NW), lambda i: (i, 0)),
        out_shape=jax.ShapeDtypeStruct((N, 3 * NW), jnp.float32),
    )(agg_t, wd_pq, wsc_q, onehot)


def _perm_cols_t(w):
    return w.reshape(-1, 32, 4).transpose(0, 2, 1).reshape(w.shape[0], 128)


def _perm_rows_dn(w):
    v = w.reshape(2, 2, 16, 2, 2, w.shape[1])
    v = jnp.transpose(v, (3, 0, 4, 1, 2, 5))
    return v.reshape(w.shape[0], w.shape[1])


def _perm_cols_q(w):
    ns = w[..., 128:].reshape(*w.shape[:-1], 64, 2)
    ns = jnp.swapaxes(ns, -1, -2).reshape(*w.shape[:-1], 128)
    return jnp.concatenate([w[..., :128], ns], axis=-1)


def _perm_rows_q(w):
    ns = w[..., 128:, :].reshape(*w.shape[:-2], 64, 2, w.shape[-1])
    ns = jnp.swapaxes(ns, -2, -3).reshape(*w.shape[:-2], 128, w.shape[-1])
    return jnp.concatenate([w[..., :128, :], ns], axis=-2)


def kernel(node_features, edge_features, edge_length_embeddings, senders,
           receivers, node_species, W_up, cg, W_r1, W_r2, W_down, W_sc):
    w_up_p = _perm_cols_t(W_up)
    cg2 = jnp.transpose(cg, (1, 2, 0)).reshape(4, 16)
    wd_pq = _perm_cols_q(_perm_rows_dn(W_down))
    wsc_q = _perm_cols_q(_perm_rows_q(W_sc))
    onehot = (node_species[:, None] == jnp.arange(4)[None, :]).astype(
        jnp.float32)

    h_t = _linear_up(node_features, w_up_p)
    p2 = _edge_prep(edge_length_embeddings, edge_features, W_r1, W_r2, cg2)
    agg2 = _make_edge_kernel()(h_t, senders.astype(jnp.int32),
                               receivers.astype(jnp.int32),
                               p2.reshape(E * 80))
    agg_t = jnp.concatenate([agg2[0, :N], agg2[1, :N]], axis=1)

    o = _finalize(agg_t, wd_pq, wsc_q, onehot)
    out = jnp.concatenate(
        [o[:, :NW],
         jnp.stack([o[:, NW:2 * NW], o[:, 2 * NW:]], axis=-1).reshape(
             N, 2 * NW)],
        axis=1)
    return out
